# baseline (speedup 1.0000x reference)
import jax
import jax.numpy as jnp
from jax.experimental import pallas as pl
from jax.experimental.pallas import tpu as pltpu

_MIDI_PITCHES = 88
_MIN_MIDI_PITCH = 21
_C = 11
_HOP = 512
_BT = 512


def _body(nc_ref, tp_ref, out_ref):
    tb = pl.program_id(1)
    nc = nc_ref[0]
    tp = tp_ref[0]
    n = nc.shape[0]

    pitch = nc[:, 0:1] - _MIN_MIDI_PITCH
    start = nc[:, 1:2] // _HOP
    end_raw = nc[:, 2:3]
    end = jnp.where(end_raw >= 0, end_raw // _HOP, -1)

    tg = tb * _BT + jax.lax.broadcasted_iota(jnp.int32, (n, _BT), 1)
    mask = ((tg >= start) & (tg < end)).astype(jnp.float32)

    q = jax.lax.broadcasted_iota(jnp.int32, (n, _MIDI_PITCHES * _C), 1)
    p_of_q = q // _C
    pm = (p_of_q == pitch).astype(jnp.float32)
    c_of_q = jax.lax.broadcasted_iota(jnp.int32, (_C, _MIDI_PITCHES * _C), 1) % _C
    s_sel = (jax.lax.broadcasted_iota(jnp.int32, (_C, _MIDI_PITCHES * _C), 0)
             == c_of_q).astype(jnp.float32)
    tpsel = jax.lax.dot_general(tp, s_sel, (((1,), (0,)), ((), ())),
                                preferred_element_type=jnp.float32)
    m_mat = pm * tpsel

    out_ref[0] = jax.lax.dot_general(
        mask, m_mat, (((0,), (0,)), ((), ())),
        preferred_element_type=jnp.float32)


def kernel(note_croppings, timbre_probs, pianorolls):
    b, n, _ = note_croppings.shape
    t_frames = pianorolls.shape[1]
    pc = _MIDI_PITCHES * _C
    out = pl.pallas_call(
        _body,
        grid=(b, t_frames // _BT),
        in_specs=[
            pl.BlockSpec((1, n, 3), lambda i, j: (i, 0, 0)),
            pl.BlockSpec((1, n, _C), lambda i, j: (i, 0, 0)),
        ],
        out_specs=pl.BlockSpec((1, _BT, pc), lambda i, j: (i, j, 0)),
        out_shape=jax.ShapeDtypeStruct((b, t_frames, pc), jnp.float32),
        compiler_params=pltpu.CompilerParams(
            dimension_semantics=("parallel", "parallel")),
    )(note_croppings, timbre_probs)
    return out.reshape(b, t_frames, _MIDI_PITCHES, _C)


# speedup vs baseline: 1.0710x; 1.0710x over previous
import jax
import jax.numpy as jnp
from jax.experimental import pallas as pl
from jax.experimental.pallas import tpu as pltpu

_MIDI_PITCHES = 88
_MIN_MIDI_PITCH = 21
_C = 11
_HOP = 512
_PC = _MIDI_PITCHES * _C


def _body(pitch_ref, start_ref, end_ref, tp_ref, out_ref):
    n = tp_ref.shape[1]
    t_frames = out_ref.shape[1]
    pitch_col = pitch_ref[0]
    start_row = start_ref[0]
    end_row = end_ref[0]
    tp = tp_ref[0]

    tg = jax.lax.broadcasted_iota(jnp.int32, (t_frames, n), 0)
    mask = ((tg >= start_row) & (tg < end_row)).astype(jnp.float32)

    p_of_q = jax.lax.broadcasted_iota(jnp.int32, (1, _PC), 1) // _C
    pm = (p_of_q == pitch_col).astype(jnp.float32)
    qc = jax.lax.broadcasted_iota(jnp.int32, (_C, _PC), 1) % _C
    s_sel = (jax.lax.broadcasted_iota(jnp.int32, (_C, _PC), 0)
             == qc).astype(jnp.float32)
    tpsel = jnp.dot(tp, s_sel, preferred_element_type=jnp.float32)
    m_mat = pm * tpsel

    out_ref[0] = jnp.dot(mask, m_mat, preferred_element_type=jnp.float32)


def kernel(note_croppings, timbre_probs, pianorolls):
    b, n, _ = note_croppings.shape
    t_frames = pianorolls.shape[1]
    pitch_col = (note_croppings[:, :, 0] - _MIN_MIDI_PITCH).reshape(b, n, 1)
    start_row = (note_croppings[:, :, 1] // _HOP).reshape(b, 1, n)
    end_raw = note_croppings[:, :, 2]
    end_row = jnp.where(end_raw >= 0, end_raw // _HOP, -1).reshape(b, 1, n)
    out = pl.pallas_call(
        _body,
        grid=(b,),
        in_specs=[
            pl.BlockSpec((1, n, 1), lambda i: (i, 0, 0)),
            pl.BlockSpec((1, 1, n), lambda i: (i, 0, 0)),
            pl.BlockSpec((1, 1, n), lambda i: (i, 0, 0)),
            pl.BlockSpec((1, n, _C), lambda i: (i, 0, 0)),
        ],
        out_specs=pl.BlockSpec((1, t_frames, _PC), lambda i: (i, 0, 0)),
        out_shape=jax.ShapeDtypeStruct((b, t_frames, _PC), jnp.float32),
        compiler_params=pltpu.CompilerParams(
            dimension_semantics=("parallel",)),
    )(pitch_col, start_row, end_row, timbre_probs)
    return out.reshape(b, t_frames, _MIDI_PITCHES, _C)


# speedup vs baseline: 5.9340x; 5.5405x over previous
import jax
import jax.numpy as jnp
from jax.experimental import pallas as pl
from jax.experimental.pallas import tpu as pltpu

_MIDI_PITCHES = 88
_MIN_MIDI_PITCH = 21
_C = 11
_HOP = 512
_PC = _MIDI_PITCHES * _C


def _body(pitch_ref, start_ref, end_ref, tpt_ref, out_ref):
    n = pitch_ref.shape[2]
    t_frames = out_ref.shape[3]
    pitch_row = pitch_ref[0]
    start_col = start_ref[0]
    end_col = end_ref[0]
    tpt = tpt_ref[0]

    tg = jax.lax.broadcasted_iota(jnp.int32, (n, t_frames), 1)
    mask = ((tg >= start_col) & (tg < end_col)).astype(jnp.float32)

    p_of_q = jax.lax.broadcasted_iota(jnp.int32, (_PC, n), 0) % _MIDI_PITCHES
    pm = (p_of_q == pitch_row).astype(jnp.float32)
    s_sel = (jax.lax.broadcasted_iota(jnp.int32, (_PC, _C), 1)
             == jax.lax.broadcasted_iota(jnp.int32, (_PC, _C), 0)
             // _MIDI_PITCHES).astype(jnp.float32)
    tpsel = jnp.dot(s_sel, tpt, preferred_element_type=jnp.float32)
    m_mat = pm * tpsel

    res = jnp.dot(m_mat, mask, preferred_element_type=jnp.float32)
    out_ref[0] = res.reshape(_C, _MIDI_PITCHES, t_frames)


def kernel(note_croppings, timbre_probs, pianorolls):
    b, n, _ = note_croppings.shape
    t_frames = pianorolls.shape[1]
    pitch_row = (note_croppings[:, :, 0] - _MIN_MIDI_PITCH).reshape(b, 1, n)
    start_col = (note_croppings[:, :, 1] // _HOP).reshape(b, n, 1)
    end_raw = note_croppings[:, :, 2]
    end_col = jnp.where(end_raw >= 0, end_raw // _HOP, -1).reshape(b, n, 1)
    tpt = timbre_probs.transpose(0, 2, 1)
    out = pl.pallas_call(
        _body,
        grid=(b,),
        in_specs=[
            pl.BlockSpec((1, 1, n), lambda i: (i, 0, 0)),
            pl.BlockSpec((1, n, 1), lambda i: (i, 0, 0)),
            pl.BlockSpec((1, n, 1), lambda i: (i, 0, 0)),
            pl.BlockSpec((1, _C, n), lambda i: (i, 0, 0)),
        ],
        out_specs=pl.BlockSpec((1, _C, _MIDI_PITCHES, t_frames),
                               lambda i: (i, 0, 0, 0)),
        out_shape=jax.ShapeDtypeStruct((b, _C, _MIDI_PITCHES, t_frames),
                                       jnp.float32),
        compiler_params=pltpu.CompilerParams(
            dimension_semantics=("parallel",)),
    )(pitch_row, start_col, end_col, tpt)
    return out.transpose(0, 3, 2, 1)


# speedup vs baseline: 6.0972x; 1.0275x over previous
import jax
import jax.numpy as jnp
from jax.experimental import pallas as pl
from jax.experimental.pallas import tpu as pltpu

_MIDI_PITCHES = 88
_MIN_MIDI_PITCH = 21
_C = 11
_HOP_SHIFT = 9
_PC = _MIDI_PITCHES * _C


def _body(nc_ref, tp_ref, out_ref):
    n = nc_ref.shape[1]
    t_frames = out_ref.shape[3]
    nc = nc_ref[0]
    tp = tp_ref[0]

    pitch_col = nc[:, 0:1] - _MIN_MIDI_PITCH
    start_col = jnp.right_shift(nc[:, 1:2], _HOP_SHIFT)
    end_raw = nc[:, 2:3]
    end_col = jnp.where(end_raw >= 0,
                        jnp.right_shift(end_raw, _HOP_SHIFT), -1)

    tg = jax.lax.broadcasted_iota(jnp.int32, (n, t_frames), 1)
    mask = ((tg >= start_col) & (tg < end_col)).astype(jnp.float32)

    q_row = jax.lax.broadcasted_iota(jnp.int32, (1, _PC), 1)
    pm = (q_row % _MIDI_PITCHES == pitch_col).astype(jnp.float32)
    s_sel = (jax.lax.broadcasted_iota(jnp.int32, (_C, _PC), 0)
             == jax.lax.broadcasted_iota(jnp.int32, (_C, _PC), 1)
             // _MIDI_PITCHES).astype(jnp.float32)
    tpsel = jnp.dot(tp, s_sel, preferred_element_type=jnp.float32)
    m_mat = pm * tpsel

    res = jax.lax.dot_general(m_mat, mask, (((0,), (0,)), ((), ())),
                              preferred_element_type=jnp.float32)
    out_ref[0] = res.reshape(_C, _MIDI_PITCHES, t_frames)


def kernel(note_croppings, timbre_probs, pianorolls):
    b, n, _ = note_croppings.shape
    t_frames = pianorolls.shape[1]
    out = pl.pallas_call(
        _body,
        grid=(b,),
        in_specs=[
            pl.BlockSpec((1, n, 3), lambda i: (i, 0, 0)),
            pl.BlockSpec((1, n, _C), lambda i: (i, 0, 0)),
        ],
        out_specs=pl.BlockSpec((1, _C, _MIDI_PITCHES, t_frames),
                               lambda i: (i, 0, 0, 0)),
        out_shape=jax.ShapeDtypeStruct((b, _C, _MIDI_PITCHES, t_frames),
                                       jnp.float32),
        compiler_params=pltpu.CompilerParams(
            dimension_semantics=("parallel",)),
    )(note_croppings, timbre_probs)
    return out.transpose(0, 3, 2, 1)
